# baseline (speedup 1.0000x reference)
import jax
import jax.numpy as jnp
from jax.experimental import pallas as pl
from jax.experimental.pallas import tpu as pltpu

_N = 10000
_NFEAT = 128
_NHID = 16
_NCLASS = 8
_BM = 200
_NB = _N // _BM
_C = 4
_NBC = _NB - _C
_NSLOT = 2 + _C


def _s1_body(x_ref, w1_ref, s1_ref):
    s1_ref[...] = jnp.dot(x_ref[...], w1_ref[...],
                          preferred_element_type=jnp.float32)


def _blk(s):
    return jnp.where(s < _NB, s, 2 * _NB - 1 - s)


def _slot(b):
    return jnp.where(b >= _NBC, b - _NBC + 2, b % 2)


def _main_body(b1_ref, w2_ref, b2_ref, s1_hbm, adj_ref, out_ref,
               buf, s1_ref, s2_ref, sem, s1_sem):
    t = pl.program_id(0)

    def issue(s):
        b = _blk(s)
        need = jnp.where(s < _NB, True, b < _NBC - 2)

        @pl.when(need)
        def _():
            pltpu.make_async_copy(
                adj_ref.at[pl.ds(b * _BM, _BM), :],
                buf.at[_slot(b)],
                sem.at[_slot(b)],
            ).start()

    @pl.when(t == 0)
    def _():
        pltpu.make_async_copy(s1_hbm, s1_ref, s1_sem).start()
        issue(0)

    @pl.when(t + 1 < 2 * _NB)
    def _():
        issue(t + 1)

    @pl.when(t == 0)
    def _():
        pltpu.make_async_copy(s1_hbm, s1_ref, s1_sem).wait()

    b = _blk(t)
    sl = _slot(b)
    need_wait = jnp.where(t < _NB, True, b < _NBC - 2)

    @pl.when(need_wait)
    def _():
        pltpu.make_async_copy(
            adj_ref.at[pl.ds(b * _BM, _BM), :],
            buf.at[sl],
            sem.at[sl],
        ).wait()

    @pl.when(t < _NB)
    def _():
        h = jnp.dot(buf[sl], s1_ref[...],
                    preferred_element_type=jnp.float32) + b1_ref[...]
        h = jnp.maximum(h, 0.0)
        s2_ref[pl.ds(b * _BM, _BM), :] = jnp.dot(
            h, w2_ref[...], preferred_element_type=jnp.float32)

    @pl.when(t >= _NB)
    def _():
        out_ref[...] = jnp.dot(buf[sl], s2_ref[...],
                               preferred_element_type=jnp.float32) + b2_ref[...]


def kernel(x, adj, W1, b1, W2, b2):
    s1 = pl.pallas_call(
        _s1_body,
        out_shape=jax.ShapeDtypeStruct((_N, _NHID), jnp.float32),
    )(x, W1)

    b1r = b1.reshape(1, _NHID)
    b2r = b2.reshape(1, _NCLASS)

    def out_map(t):
        return (jnp.where(t < _NB, _NB - 1, 2 * _NB - 1 - t), 0)

    return pl.pallas_call(
        _main_body,
        grid=(2 * _NB,),
        in_specs=[
            pl.BlockSpec((1, _NHID), lambda t: (0, 0)),
            pl.BlockSpec((_NHID, _NCLASS), lambda t: (0, 0)),
            pl.BlockSpec((1, _NCLASS), lambda t: (0, 0)),
            pl.BlockSpec(memory_space=pltpu.MemorySpace.HBM),
            pl.BlockSpec(memory_space=pltpu.MemorySpace.HBM),
        ],
        out_specs=pl.BlockSpec((_BM, _NCLASS), out_map),
        out_shape=jax.ShapeDtypeStruct((_N, _NCLASS), jnp.float32),
        scratch_shapes=[
            pltpu.VMEM((_NSLOT, _BM, _N), jnp.float32),
            pltpu.VMEM((_N, _NHID), jnp.float32),
            pltpu.VMEM((_N, _NCLASS), jnp.float32),
            pltpu.SemaphoreType.DMA((_NSLOT,)),
            pltpu.SemaphoreType.DMA,
        ],
        compiler_params=pltpu.CompilerParams(
            vmem_limit_bytes=64 * 1024 * 1024,
        ),
    )(b1r, W2, b2r, s1, adj)


# speedup vs baseline: 1.0343x; 1.0343x over previous
import jax
import jax.numpy as jnp
from jax.experimental import pallas as pl
from jax.experimental.pallas import tpu as pltpu

_N = 10000
_NFEAT = 128
_NHID = 16
_NCLASS = 8
_BM = 400
_NB = _N // _BM


def _s1_body(x_ref, w1_ref, s1_ref):
    s1_ref[...] = jnp.dot(x_ref[...], w1_ref[...],
                          preferred_element_type=jnp.float32)


def _blk(s):
    return jnp.where(s < _NB, s, 2 * _NB - 1 - s)


def _slot(b):
    return jnp.where(b == _NB - 1, 2, b % 2)


def _main_body(b1_ref, w2_ref, b2_ref, s1_hbm, adj_ref, out_ref,
               buf, s1_ref, s2_ref, sem, s1_sem):
    t = pl.program_id(0)

    def issue(s):
        b = _blk(s)
        sl = _slot(b)

        need = jnp.where(s < _NB, True, b < _NB - 3)

        @pl.when(need)
        def _():
            pltpu.make_async_copy(
                adj_ref.at[pl.ds(b * _BM, _BM), :],
                buf.at[sl], sem.at[sl]).start()

    @pl.when(t == 0)
    def _():
        pltpu.make_async_copy(s1_hbm, s1_ref, s1_sem).start()
        issue(0)

    @pl.when(t + 1 < 2 * _NB)
    def _():
        issue(t + 1)

    @pl.when(t == 0)
    def _():
        pltpu.make_async_copy(s1_hbm, s1_ref, s1_sem).wait()

    b = _blk(t)
    sl = _slot(b)

    need_wait = jnp.where(t < _NB, True, b < _NB - 3)

    @pl.when(need_wait)
    def _():
        pltpu.make_async_copy(
            adj_ref.at[pl.ds(b * _BM, _BM), :],
            buf.at[sl], sem.at[sl]).wait()

    @pl.when(t < _NB)
    def _():
        h = jnp.dot(buf[sl], s1_ref[...],
                    preferred_element_type=jnp.float32) + b1_ref[...]
        h = jnp.maximum(h, 0.0)
        s2_ref[pl.ds(b * _BM, _BM), :] = jnp.dot(
            h, w2_ref[...], preferred_element_type=jnp.float32)

    @pl.when(t >= _NB)
    def _():
        out_ref[...] = jnp.dot(buf[sl], s2_ref[...],
                               preferred_element_type=jnp.float32) + b2_ref[...]


def kernel(x, adj, W1, b1, W2, b2):
    s1 = pl.pallas_call(
        _s1_body,
        out_shape=jax.ShapeDtypeStruct((_N, _NHID), jnp.float32),
    )(x, W1)

    b1r = b1.reshape(1, _NHID)
    b2r = b2.reshape(1, _NCLASS)

    def out_map(t):
        return (jnp.where(t < _NB, _NB - 1, 2 * _NB - 1 - t), 0)

    return pl.pallas_call(
        _main_body,
        grid=(2 * _NB,),
        in_specs=[
            pl.BlockSpec((1, _NHID), lambda t: (0, 0)),
            pl.BlockSpec((_NHID, _NCLASS), lambda t: (0, 0)),
            pl.BlockSpec((1, _NCLASS), lambda t: (0, 0)),
            pl.BlockSpec(memory_space=pltpu.MemorySpace.HBM),
            pl.BlockSpec(memory_space=pltpu.MemorySpace.HBM),
        ],
        out_specs=pl.BlockSpec((_BM, _NCLASS), out_map),
        out_shape=jax.ShapeDtypeStruct((_N, _NCLASS), jnp.float32),
        scratch_shapes=[
            pltpu.VMEM((3, _BM, _N), jnp.float32),
            pltpu.VMEM((_N, _NHID), jnp.float32),
            pltpu.VMEM((_N, _NCLASS), jnp.float32),
            pltpu.SemaphoreType.DMA((3,)),
            pltpu.SemaphoreType.DMA,
        ],
        compiler_params=pltpu.CompilerParams(
            vmem_limit_bytes=64 * 1024 * 1024,
        ),
    )(b1r, W2, b2r, s1, adj)
